# baseline (speedup 1.0000x reference)
import math

import jax
import jax.numpy as jnp
from jax.experimental import pallas as pl
from jax.experimental.pallas import tpu as pltpu

TOP_K = 2
N_EXP = 64
CAP_FACTOR = 1.25
MIN_CAP = 4


def _capacity(num_tokens: int) -> int:
    cap = math.floor(TOP_K * CAP_FACTOR * num_tokens / N_EXP)
    cap += cap % 2
    return int(max(cap, MIN_CAP))


def _router_stage1(x2d, W_g, bn):
    N, D = x2d.shape
    E = N_EXP
    nb = N // bn
    cap = _capacity(N)

    def body(x_ref, wg_ref, e0_ref, e1_ref, p0_ref, p1_ref, r0_ref, r1p_ref,
             cnt_ref, used_ref, c0_s, c1_s):
        i = pl.program_id(0)

        @pl.when(i == 0)
        def _():
            c0_s[...] = jnp.zeros_like(c0_s)
            c1_s[...] = jnp.zeros_like(c1_s)

        lt = jax.lax.dot_general(
            wg_ref[...], x_ref[...], (((1,), (1,)), ((), ())),
            preferred_element_type=jnp.float32)
        iota_e = jax.lax.broadcasted_iota(jnp.int32, (E, bn), 0)
        m0 = jnp.max(lt, axis=0, keepdims=True)
        e0 = jnp.min(jnp.where(lt == m0, iota_e, E), axis=0, keepdims=True)
        h0 = iota_e == e0
        l2 = jnp.where(h0, -jnp.inf, lt)
        m1 = jnp.max(l2, axis=0, keepdims=True)
        e1 = jnp.min(jnp.where(l2 == m1, iota_e, E), axis=0, keepdims=True)
        h1 = iota_e == e1
        d = jnp.exp(m1 - m0)
        s = 1.0 + d
        p0 = 1.0 / s
        p1 = d / s

        h0f = h0.astype(jnp.float32)
        h1f = h1.astype(jnp.float32)
        ri = jax.lax.broadcasted_iota(jnp.int32, (bn, bn), 0)
        ci = jax.lax.broadcasted_iota(jnp.int32, (bn, bn), 1)
        ltri = (ri < ci).astype(jnp.float32)
        excl0 = jax.lax.dot_general(h0f, ltri, (((1,), (0,)), ((), ())),
                                    preferred_element_type=jnp.float32)
        excl1 = jax.lax.dot_general(h1f, ltri, (((1,), (0,)), ((), ())),
                                    preferred_element_type=jnp.float32)
        base0 = c0_s[...]
        base1 = c1_s[...]
        r0 = jnp.sum((excl0 + base0) * h0f, axis=0, keepdims=True)
        r1p = jnp.sum((excl1 + base1) * h1f, axis=0, keepdims=True)
        new0 = base0 + jnp.sum(h0f, axis=1, keepdims=True)
        new1 = base1 + jnp.sum(h1f, axis=1, keepdims=True)
        c0_s[...] = new0
        c1_s[...] = new1

        e0_ref[...] = e0
        e1_ref[...] = e1
        p0_ref[...] = p0
        p1_ref[...] = p1
        r0_ref[...] = r0.astype(jnp.int32)
        r1p_ref[...] = r1p.astype(jnp.int32)
        cnt_ref[...] = new0.astype(jnp.int32)
        used_ref[...] = jnp.minimum(new0 + new1, float(cap)).astype(jnp.int32)

    out_shapes = (
        jax.ShapeDtypeStruct((1, N), jnp.int32),
        jax.ShapeDtypeStruct((1, N), jnp.int32),
        jax.ShapeDtypeStruct((1, N), jnp.float32),
        jax.ShapeDtypeStruct((1, N), jnp.float32),
        jax.ShapeDtypeStruct((1, N), jnp.int32),
        jax.ShapeDtypeStruct((1, N), jnp.int32),
        jax.ShapeDtypeStruct((E, 1), jnp.int32),
        jax.ShapeDtypeStruct((E, 1), jnp.int32),
    )
    tok_spec = pl.BlockSpec((1, bn), lambda i: (0, i))
    col_spec = pl.BlockSpec((E, 1), lambda i: (0, 0))
    return pl.pallas_call(
        body,
        grid=(nb,),
        in_specs=[
            pl.BlockSpec((bn, D), lambda i: (i, 0)),
            pl.BlockSpec((E, D), lambda i: (0, 0)),
        ],
        out_specs=(
            tok_spec, tok_spec, tok_spec, tok_spec, tok_spec, tok_spec,
            col_spec, col_spec,
        ),
        out_shape=out_shapes,
        scratch_shapes=[
            pltpu.VMEM((E, 1), jnp.float32),
            pltpu.VMEM((E, 1), jnp.float32),
        ],
    )(x2d, W_g)


def _dispatch_stage2(e0, e1, p0, p1, r0, r1p, cnt0, N, cap, bn):
    E = N_EXP
    F = E * cap
    nb = N // bn

    def targets(e0_ref, e1_ref, p0_ref, p1_ref, r0_ref, r1p_ref, cnt_ref):
        iota_e = jax.lax.broadcasted_iota(jnp.int32, (E, bn), 0)
        cnt = cnt_ref[...]
        h1 = iota_e == e1_ref[...]
        add1 = jnp.sum(jnp.where(h1, cnt, 0), axis=0, keepdims=True)
        r0v = r0_ref[...]
        r1v = r1p_ref[...] + add1
        p0 = p0_ref[...]
        p1 = p1_ref[...]
        t0 = jnp.where(r0v < cap, e0_ref[...] * cap + r0v, -1)
        t1 = jnp.where(r1v < cap, e1_ref[...] * cap + r1v, -1)
        t0 = jnp.where(p0 != 0.0, t0, -1)
        t1 = jnp.where(p1 != 0.0, t1, -1)
        return t0, t1, p0, p1

    def body(e0_ref, e1_ref, p0_ref, p1_ref, r0_ref, r1p_ref, cnt_ref,
             cb_ref, t0_ref, t1_ref):
        t0, t1, p0, p1 = targets(e0_ref, e1_ref, p0_ref, p1_ref, r0_ref,
                                 r1p_ref, cnt_ref)
        f = jax.lax.broadcasted_iota(jnp.int32, (F, bn), 0)
        cb_ref[...] = jnp.where(f == t0, p0, jnp.where(f == t1, p1, 0.0))
        t0_ref[...] = t0
        t1_ref[...] = t1

    tok_spec = pl.BlockSpec((1, bn), lambda i: (0, i))
    out_spec = pl.BlockSpec((F, bn), lambda i: (0, i))
    in_specs = [tok_spec, tok_spec, tok_spec, tok_spec, tok_spec, tok_spec,
                pl.BlockSpec((E, 1), lambda i: (0, 0))]
    return pl.pallas_call(
        body,
        grid=(nb,),
        in_specs=in_specs,
        out_specs=(out_spec, tok_spec, tok_spec),
        out_shape=(
            jax.ShapeDtypeStruct((F, N), jnp.float32),
            jax.ShapeDtypeStruct((1, N), jnp.int32),
            jax.ShapeDtypeStruct((1, N), jnp.int32),
        ),
    )(e0, e1, p0, p1, r0, r1p, cnt0)


def kernel(x, W_g):
    B, T, D = x.shape
    N = B * T
    cap = _capacity(N)
    x2d = x.reshape(N, D)
    e0, e1, p0, p1, r0, r1p, cnt0, used = _router_stage1(x2d, W_g, bn=256)
    cb2, t0x, t1x = _dispatch_stage2(e0, e1, p0, p1, r0, r1p, cnt0, N, cap,
                                     bn=512)
    cb = cb2.reshape(N_EXP, cap, N).transpose(2, 0, 1)
    f = jax.lax.broadcasted_iota(jnp.int32, (N_EXP * cap, N), 0)
    m2 = (f == t0x) | (f == t1x)
    mask = m2.reshape(N_EXP, cap, N).transpose(2, 0, 1)
    return (used.reshape(N_EXP), cb, mask)


# speedup vs baseline: 1.0308x; 1.0308x over previous
import math

import jax
import jax.numpy as jnp
from jax.experimental import pallas as pl
from jax.experimental.pallas import tpu as pltpu

TOP_K = 2
N_EXP = 64
CAP_FACTOR = 1.25
MIN_CAP = 4


def _capacity(num_tokens: int) -> int:
    cap = math.floor(TOP_K * CAP_FACTOR * num_tokens / N_EXP)
    cap += cap % 2
    return int(max(cap, MIN_CAP))


def _router(x2d, W_g, bn):
    N, D = x2d.shape
    E = N_EXP
    cap = _capacity(N)
    F = E * cap
    nb = N // bn

    def body(x_ref, wg_ref, cb_ref, t0_ref, t1_ref, used_ref,
             c0_s, c1_s, t0_s, e1_s, r1p_s, p0_s, p1_s):
        k = pl.program_id(0)
        i = pl.program_id(1)

        @pl.when((k == 0) & (i == 0))
        def _():
            c0_s[...] = jnp.zeros_like(c0_s)
            c1_s[...] = jnp.zeros_like(c1_s)

        @pl.when(k == 0)
        def _():
            lt = jax.lax.dot_general(
                wg_ref[...], x_ref[...], (((1,), (1,)), ((), ())),
                preferred_element_type=jnp.float32)
            iota_e = jax.lax.broadcasted_iota(jnp.int32, (E, bn), 0)
            m0 = jnp.max(lt, axis=0, keepdims=True)
            e0 = jnp.min(jnp.where(lt == m0, iota_e, E), axis=0,
                         keepdims=True)
            h0 = iota_e == e0
            l2 = jnp.where(h0, -jnp.inf, lt)
            m1 = jnp.max(l2, axis=0, keepdims=True)
            e1 = jnp.min(jnp.where(l2 == m1, iota_e, E), axis=0,
                         keepdims=True)
            h1 = iota_e == e1
            d = jnp.exp(m1 - m0)
            s = 1.0 + d
            p0 = 1.0 / s
            p1 = d / s

            h0f = h0.astype(jnp.float32)
            h1f = h1.astype(jnp.float32)
            ri = jax.lax.broadcasted_iota(jnp.int32, (bn, bn), 0)
            ci = jax.lax.broadcasted_iota(jnp.int32, (bn, bn), 1)
            ltri = (ri < ci).astype(jnp.float32)
            excl0 = jax.lax.dot_general(h0f, ltri, (((1,), (0,)), ((), ())),
                                        preferred_element_type=jnp.float32)
            excl1 = jax.lax.dot_general(h1f, ltri, (((1,), (0,)), ((), ())),
                                        preferred_element_type=jnp.float32)
            base0 = c0_s[...]
            base1 = c1_s[...]
            r0 = jnp.sum((excl0 + base0) * h0f, axis=0, keepdims=True)
            r1p = jnp.sum((excl1 + base1) * h1f, axis=0, keepdims=True)
            new0 = base0 + jnp.sum(h0f, axis=1, keepdims=True)
            new1 = base1 + jnp.sum(h1f, axis=1, keepdims=True)
            c0_s[...] = new0
            c1_s[...] = new1

            r0i = r0.astype(jnp.int32)
            t0 = jnp.where((r0i < cap) & (p0 != 0.0), e0 * cap + r0i, -1)
            t0_s[pl.ds(i, 1), :] = t0
            e1_s[pl.ds(i, 1), :] = e1
            r1p_s[pl.ds(i, 1), :] = r1p.astype(jnp.int32)
            p0_s[pl.ds(i, 1), :] = p0
            p1_s[pl.ds(i, 1), :] = p1
            used_ref[...] = jnp.minimum(new0 + new1, float(cap)).astype(
                jnp.int32)

        @pl.when(k == 1)
        def _():
            iota_e = jax.lax.broadcasted_iota(jnp.int32, (E, bn), 0)
            e1 = e1_s[pl.ds(i, 1), :]
            h1 = iota_e == e1
            add1 = jnp.sum(jnp.where(h1, c0_s[...], 0.0), axis=0,
                           keepdims=True)
            r1 = r1p_s[pl.ds(i, 1), :] + add1.astype(jnp.int32)
            p0 = p0_s[pl.ds(i, 1), :]
            p1 = p1_s[pl.ds(i, 1), :]
            t0 = t0_s[pl.ds(i, 1), :]
            t1 = jnp.where((r1 < cap) & (p1 != 0.0), e1 * cap + r1, -1)
            f = jax.lax.broadcasted_iota(jnp.int32, (F, bn), 0)
            cb_ref[...] = jnp.where(f == t0, p0, jnp.where(f == t1, p1, 0.0))
            t0_ref[...] = t0
            t1_ref[...] = t1

    out_spec = pl.BlockSpec((F, bn), lambda k, i: (0, i * k))
    tok_spec = pl.BlockSpec((1, bn), lambda k, i: (0, i * k))
    return pl.pallas_call(
        body,
        grid=(2, nb),
        in_specs=[
            pl.BlockSpec((bn, D), lambda k, i: (i * (1 - k), 0)),
            pl.BlockSpec((E, D), lambda k, i: (0, 0)),
        ],
        out_specs=(
            out_spec, tok_spec, tok_spec,
            pl.BlockSpec((E, 1), lambda k, i: (0, 0)),
        ),
        out_shape=(
            jax.ShapeDtypeStruct((F, N), jnp.float32),
            jax.ShapeDtypeStruct((1, N), jnp.int32),
            jax.ShapeDtypeStruct((1, N), jnp.int32),
            jax.ShapeDtypeStruct((E, 1), jnp.int32),
        ),
        scratch_shapes=[
            pltpu.VMEM((E, 1), jnp.float32),
            pltpu.VMEM((E, 1), jnp.float32),
            pltpu.VMEM((nb, bn), jnp.int32),
            pltpu.VMEM((nb, bn), jnp.int32),
            pltpu.VMEM((nb, bn), jnp.int32),
            pltpu.VMEM((nb, bn), jnp.float32),
            pltpu.VMEM((nb, bn), jnp.float32),
        ],
    )(x2d, W_g)


def kernel(x, W_g):
    B, T, D = x.shape
    N = B * T
    cap = _capacity(N)
    x2d = x.reshape(N, D)
    cb2, t0x, t1x, used = _router(x2d, W_g, bn=256)
    cb = cb2.reshape(N_EXP, cap, N).transpose(2, 0, 1)
    f = jax.lax.broadcasted_iota(jnp.int32, (N_EXP * cap, N), 0)
    m2 = (f == t0x) | (f == t1x)
    mask = m2.reshape(N_EXP, cap, N).transpose(2, 0, 1)
    return (used.reshape(N_EXP), cb, mask)


# speedup vs baseline: 1.0808x; 1.0485x over previous
import math

import jax
import jax.numpy as jnp
from jax.experimental import pallas as pl
from jax.experimental.pallas import tpu as pltpu

TOP_K = 2
N_EXP = 64
CAP_FACTOR = 1.25
MIN_CAP = 4


def _capacity(num_tokens: int) -> int:
    cap = math.floor(TOP_K * CAP_FACTOR * num_tokens / N_EXP)
    cap += cap % 2
    return int(max(cap, MIN_CAP))


def _router(x2d, W_g, bn):
    N, D = x2d.shape
    E = N_EXP
    cap = _capacity(N)
    F = E * cap
    nb = N // bn

    def body(x_ref, wg_ref, cb_ref, t0_ref, t1_ref, used_ref,
             c0_s, c1_s, t0_s, e1_s, r1p_s, p0_s, p1_s):
        k = pl.program_id(0)
        i = pl.program_id(1)

        @pl.when((k == 0) & (i == 0))
        def _():
            c0_s[...] = jnp.zeros_like(c0_s)
            c1_s[...] = jnp.zeros_like(c1_s)

        @pl.when(k == 0)
        def _():
            lt = jax.lax.dot_general(
                wg_ref[...], x_ref[...], (((1,), (1,)), ((), ())),
                preferred_element_type=jnp.float32)
            iota_e = jax.lax.broadcasted_iota(jnp.int32, (E, bn), 0)
            m0 = jnp.max(lt, axis=0, keepdims=True)
            e0 = jnp.min(jnp.where(lt == m0, iota_e, E), axis=0,
                         keepdims=True)
            h0 = iota_e == e0
            l2 = jnp.where(h0, -jnp.inf, lt)
            m1 = jnp.max(l2, axis=0, keepdims=True)
            e1 = jnp.min(jnp.where(l2 == m1, iota_e, E), axis=0,
                         keepdims=True)
            h1 = iota_e == e1
            d = jnp.exp(m1 - m0)
            s = 1.0 + d
            p0 = 1.0 / s
            p1 = d / s

            h0f = h0.astype(jnp.float32)
            h1f = h1.astype(jnp.float32)
            ri = jax.lax.broadcasted_iota(jnp.int32, (bn, bn), 0)
            ci = jax.lax.broadcasted_iota(jnp.int32, (bn, bn), 1)
            ltri = (ri < ci).astype(jnp.float32)
            excl0 = jax.lax.dot_general(h0f, ltri, (((1,), (0,)), ((), ())),
                                        preferred_element_type=jnp.float32)
            excl1 = jax.lax.dot_general(h1f, ltri, (((1,), (0,)), ((), ())),
                                        preferred_element_type=jnp.float32)
            base0 = c0_s[...]
            base1 = c1_s[...]
            r0 = jnp.sum((excl0 + base0) * h0f, axis=0, keepdims=True)
            r1p = jnp.sum((excl1 + base1) * h1f, axis=0, keepdims=True)
            new0 = base0 + jnp.sum(h0f, axis=1, keepdims=True)
            new1 = base1 + jnp.sum(h1f, axis=1, keepdims=True)
            c0_s[...] = new0
            c1_s[...] = new1

            r0i = r0.astype(jnp.int32)
            t0 = jnp.where((r0i < cap) & (p0 != 0.0), e0 * cap + r0i, -1)
            t0_s[pl.ds(i, 1), :] = t0
            e1_s[pl.ds(i, 1), :] = e1
            r1p_s[pl.ds(i, 1), :] = r1p.astype(jnp.int32)
            p0_s[pl.ds(i, 1), :] = p0
            p1_s[pl.ds(i, 1), :] = p1
            used_ref[...] = jnp.minimum(new0 + new1, float(cap)).astype(
                jnp.int32)

        @pl.when(k == 1)
        def _():
            iota_e = jax.lax.broadcasted_iota(jnp.int32, (E, bn), 0)
            e1 = e1_s[pl.ds(i, 1), :]
            h1 = iota_e == e1
            add1 = jnp.sum(jnp.where(h1, c0_s[...], 0.0), axis=0,
                           keepdims=True)
            r1 = r1p_s[pl.ds(i, 1), :] + add1.astype(jnp.int32)
            p0 = p0_s[pl.ds(i, 1), :]
            p1 = p1_s[pl.ds(i, 1), :]
            t0 = t0_s[pl.ds(i, 1), :]
            t1 = jnp.where((r1 < cap) & (p1 != 0.0), e1 * cap + r1, -1)
            f = jax.lax.broadcasted_iota(jnp.int32, (F, bn), 0)
            cb_ref[...] = jnp.where(f == t0, p0, jnp.where(f == t1, p1, 0.0))
            t0_ref[...] = t0
            t1_ref[...] = t1

    out_spec = pl.BlockSpec((F, bn), lambda k, i: (0, i * k))
    tok_spec = pl.BlockSpec((1, bn), lambda k, i: (0, i * k))
    return pl.pallas_call(
        body,
        grid=(2, nb),
        in_specs=[
            pl.BlockSpec((bn, D), lambda k, i: (i * (1 - k), 0)),
            pl.BlockSpec((E, D), lambda k, i: (0, 0)),
        ],
        out_specs=(
            out_spec, tok_spec, tok_spec,
            pl.BlockSpec((E, 1), lambda k, i: (0, 0)),
        ),
        out_shape=(
            jax.ShapeDtypeStruct((F, N), jnp.float32),
            jax.ShapeDtypeStruct((1, N), jnp.int32),
            jax.ShapeDtypeStruct((1, N), jnp.int32),
            jax.ShapeDtypeStruct((E, 1), jnp.int32),
        ),
        scratch_shapes=[
            pltpu.VMEM((E, 1), jnp.float32),
            pltpu.VMEM((E, 1), jnp.float32),
            pltpu.VMEM((nb, bn), jnp.int32),
            pltpu.VMEM((nb, bn), jnp.int32),
            pltpu.VMEM((nb, bn), jnp.int32),
            pltpu.VMEM((nb, bn), jnp.float32),
            pltpu.VMEM((nb, bn), jnp.float32),
        ],
    )(x2d, W_g)


def kernel(x, W_g):
    B, T, D = x.shape
    N = B * T
    cap = _capacity(N)
    x2d = x.reshape(N, D)
    cb2, t0x, t1x, used = _router(x2d, W_g, bn=512)
    cb = cb2.reshape(N_EXP, cap, N).transpose(2, 0, 1)
    f = jax.lax.broadcasted_iota(jnp.int32, (N_EXP * cap, N), 0)
    m2 = (f == t0x) | (f == t1x)
    mask = m2.reshape(N_EXP, cap, N).transpose(2, 0, 1)
    return (used.reshape(N_EXP), cb, mask)


# speedup vs baseline: 1.1196x; 1.0359x over previous
import math

import jax
import jax.numpy as jnp
from jax.experimental import pallas as pl
from jax.experimental.pallas import tpu as pltpu

TOP_K = 2
N_EXP = 64
CAP_FACTOR = 1.25
MIN_CAP = 4


def _capacity(num_tokens: int) -> int:
    cap = math.floor(TOP_K * CAP_FACTOR * num_tokens / N_EXP)
    cap += cap % 2
    return int(max(cap, MIN_CAP))


def _router(x2d, W_g, bn):
    N, D = x2d.shape
    E = N_EXP
    cap = _capacity(N)
    F = E * cap
    nb = N // bn

    def body(x_ref, wg_ref, cb_ref, t0_ref, t1_ref, used_ref,
             c0_s, c1_s, t0_s, e1_s, r1p_s, p0_s, p1_s):
        k = pl.program_id(0)
        i = pl.program_id(1)

        @pl.when((k == 0) & (i == 0))
        def _():
            c0_s[...] = jnp.zeros_like(c0_s)
            c1_s[...] = jnp.zeros_like(c1_s)

        @pl.when(k == 0)
        def _():
            lt = jax.lax.dot_general(
                wg_ref[...], x_ref[...], (((1,), (1,)), ((), ())),
                preferred_element_type=jnp.float32)
            iota_e = jax.lax.broadcasted_iota(jnp.int32, (E, bn), 0)
            m0 = jnp.max(lt, axis=0, keepdims=True)
            e0 = jnp.min(jnp.where(lt == m0, iota_e, E), axis=0,
                         keepdims=True)
            h0 = iota_e == e0
            l2 = jnp.where(h0, -jnp.inf, lt)
            m1 = jnp.max(l2, axis=0, keepdims=True)
            e1 = jnp.min(jnp.where(l2 == m1, iota_e, E), axis=0,
                         keepdims=True)
            h1 = iota_e == e1
            d = jnp.exp(m1 - m0)
            s = 1.0 + d
            p0 = 1.0 / s
            p1 = d / s

            h0f = h0.astype(jnp.float32)
            h1f = h1.astype(jnp.float32)
            ri = jax.lax.broadcasted_iota(jnp.int32, (bn, bn), 0)
            ci = jax.lax.broadcasted_iota(jnp.int32, (bn, bn), 1)
            ltri = (ri < ci).astype(jnp.float32)
            excl0 = jax.lax.dot_general(h0f, ltri, (((1,), (0,)), ((), ())),
                                        preferred_element_type=jnp.float32)
            excl1 = jax.lax.dot_general(h1f, ltri, (((1,), (0,)), ((), ())),
                                        preferred_element_type=jnp.float32)
            base0 = c0_s[...]
            base1 = c1_s[...]
            r0 = jnp.sum((excl0 + base0) * h0f, axis=0, keepdims=True)
            r1p = jnp.sum((excl1 + base1) * h1f, axis=0, keepdims=True)
            new0 = base0 + jnp.sum(h0f, axis=1, keepdims=True)
            new1 = base1 + jnp.sum(h1f, axis=1, keepdims=True)
            c0_s[...] = new0
            c1_s[...] = new1

            r0i = r0.astype(jnp.int32)
            t0 = jnp.where((r0i < cap) & (p0 != 0.0), e0 * cap + r0i, -1)
            t0_s[pl.ds(i, 1), :] = t0
            e1_s[pl.ds(i, 1), :] = e1
            r1p_s[pl.ds(i, 1), :] = r1p.astype(jnp.int32)
            p0_s[pl.ds(i, 1), :] = p0
            p1_s[pl.ds(i, 1), :] = p1
            used_ref[...] = jnp.minimum(new0 + new1, float(cap)).astype(
                jnp.int32)

        @pl.when(k == 1)
        def _():
            iota_e = jax.lax.broadcasted_iota(jnp.int32, (E, bn), 0)
            e1 = e1_s[pl.ds(i, 1), :]
            h1 = iota_e == e1
            add1 = jnp.sum(jnp.where(h1, c0_s[...], 0.0), axis=0,
                           keepdims=True)
            r1 = r1p_s[pl.ds(i, 1), :] + add1.astype(jnp.int32)
            p0 = p0_s[pl.ds(i, 1), :]
            p1 = p1_s[pl.ds(i, 1), :]
            t0 = t0_s[pl.ds(i, 1), :]
            t1 = jnp.where((r1 < cap) & (p1 != 0.0), e1 * cap + r1, -1)
            f = jax.lax.broadcasted_iota(jnp.int32, (F, bn), 0)
            cb_ref[...] = jnp.where(f == t0, p0, jnp.where(f == t1, p1, 0.0))
            t0_ref[...] = t0
            t1_ref[...] = t1

    out_spec = pl.BlockSpec((F, bn), lambda k, i: (0, i * k))
    tok_spec = pl.BlockSpec((1, bn), lambda k, i: (0, i * k))
    return pl.pallas_call(
        body,
        grid=(2, nb),
        in_specs=[
            pl.BlockSpec((bn, D), lambda k, i: (i * (1 - k) + k * (nb - 1), 0)),
            pl.BlockSpec((E, D), lambda k, i: (0, 0)),
        ],
        out_specs=(
            out_spec, tok_spec, tok_spec,
            pl.BlockSpec((E, 1), lambda k, i: (0, 0)),
        ),
        out_shape=(
            jax.ShapeDtypeStruct((F, N), jnp.float32),
            jax.ShapeDtypeStruct((1, N), jnp.int32),
            jax.ShapeDtypeStruct((1, N), jnp.int32),
            jax.ShapeDtypeStruct((E, 1), jnp.int32),
        ),
        scratch_shapes=[
            pltpu.VMEM((E, 1), jnp.float32),
            pltpu.VMEM((E, 1), jnp.float32),
            pltpu.VMEM((nb, bn), jnp.int32),
            pltpu.VMEM((nb, bn), jnp.int32),
            pltpu.VMEM((nb, bn), jnp.int32),
            pltpu.VMEM((nb, bn), jnp.float32),
            pltpu.VMEM((nb, bn), jnp.float32),
        ],
    )(x2d, W_g)


def kernel(x, W_g):
    B, T, D = x.shape
    N = B * T
    cap = _capacity(N)
    x2d = x.reshape(N, D)
    cb2, t0x, t1x, used = _router(x2d, W_g, bn=512)
    cb = cb2.reshape(N_EXP, cap, N).transpose(2, 0, 1)
    f = jax.lax.broadcasted_iota(jnp.int32, (N_EXP * cap, N), 0)
    m2 = (f == t0x) | (f == t1x)
    mask = m2.reshape(N_EXP, cap, N).transpose(2, 0, 1)
    return (used.reshape(N_EXP), cb, mask)
